# baseline (speedup 1.0000x reference)
import functools

import jax
import jax.numpy as jnp
from jax import lax
from jax.experimental import pallas as pl
from jax.experimental.pallas import tpu as pltpu
from jax.experimental.pallas import tpu_sc as plsc

_N = 100000
_NUM_SP = 1024
_CHUNK = 3136
_LAST_BASE = _N - _CHUNK
_NUM_CORES = 2
_GROUPS = _CHUNK // 16
_UNROLL = 4


def _gather_body(tab_t, tab_r, idx, out_t, out_r,
                 tab_t_v, tab_r_v, idx_v, out_t_v, out_r_v, sem):
    wid = lax.axis_index("s") * _NUM_CORES + lax.axis_index("c")
    base = jnp.minimum(wid * _CHUNK, _LAST_BASE)

    ct = pltpu.async_copy(tab_t, tab_t_v, sem)
    cr = pltpu.async_copy(tab_r, tab_r_v, sem)
    pltpu.sync_copy(idx.at[pl.ds(base, _CHUNK)], idx_v)
    ct.wait()
    cr.wait()

    lanes3 = lax.iota(jnp.int32, 16) * 3

    def group(g):
        sp3 = idx_v[pl.ds(g * 16, 16)] * 3
        obase = g * 48 + lanes3
        for tab_v, out_v in ((tab_t_v, out_t_v), (tab_r_v, out_r_v)):
            for c in range(3):
                vals = plsc.load_gather(tab_v, [sp3 + c])
                plsc.store_scatter(out_v, [obase + c], vals)

    def body(i, carry):
        for u in range(_UNROLL):
            group(i * _UNROLL + u)
        return carry

    lax.fori_loop(0, _GROUPS // _UNROLL, body, 0)

    pltpu.sync_copy(out_t_v, out_t.at[pl.ds(base * 3, _CHUNK * 3)])
    pltpu.sync_copy(out_r_v, out_r.at[pl.ds(base * 3, _CHUNK * 3)])


def kernel(sp_delta_t, sp_delta_r, p2sp):
    mesh = plsc.VectorSubcoreMesh(core_axis_name="c", subcore_axis_name="s")
    run = pl.kernel(
        _gather_body,
        mesh=mesh,
        compiler_params=pltpu.CompilerParams(needs_layout_passes=False),
        out_type=(
            jax.ShapeDtypeStruct((_N * 3,), jnp.float32),
            jax.ShapeDtypeStruct((_N * 3,), jnp.float32),
        ),
        scratch_types=[
            pltpu.VMEM((_NUM_SP * 3,), jnp.float32),
            pltpu.VMEM((_NUM_SP * 3,), jnp.float32),
            pltpu.VMEM((_CHUNK,), jnp.int32),
            pltpu.VMEM((_CHUNK * 3,), jnp.float32),
            pltpu.VMEM((_CHUNK * 3,), jnp.float32),
            pltpu.SemaphoreType.DMA,
        ],
    )
    flat_t, flat_r = run(sp_delta_t.reshape(-1), sp_delta_r.reshape(-1), p2sp)
    return flat_t.reshape(_N, 3), flat_r.reshape(_N, 3)


# speedup vs baseline: 6.7502x; 6.7502x over previous
import functools

import jax
import jax.numpy as jnp
from jax import lax
from jax.experimental import pallas as pl
from jax.experimental.pallas import tpu as pltpu
from jax.experimental.pallas import tpu_sc as plsc

_N = 100000
_NUM_SP = 1024
_LANE = 128
_TILES = 782
_OUT_FLAT = _TILES * 4 * _LANE
_NUM_CORES = 2
_NW = 32
_TPW = 25
_CHUNK = _TPW * _LANE
_WSTAGE = _TPW * 4 * _LANE
_LAST_BASE = _TILES * _LANE - _CHUNK
_LAST_VALID = _N - _LAST_BASE


def _gather_body(tab_t, tab_r, idx, out_t, out_r,
                 tab_t_v, tab_r_v, idx_v, out_t_v, out_r_v, sem):
    wid = lax.axis_index("s") * _NUM_CORES + lax.axis_index("c")
    is_tail = wid == _NW - 1
    base = jnp.where(is_tail, _LAST_BASE, wid * _CHUNK)

    ct = pltpu.async_copy(tab_t, tab_t_v, sem)
    cr = pltpu.async_copy(tab_r, tab_r_v, sem)

    @pl.when(jnp.logical_not(is_tail))
    def _():
        pltpu.sync_copy(idx.at[pl.ds(base, _CHUNK)], idx_v.at[pl.ds(0, _CHUNK)])

    @pl.when(is_tail)
    def _():
        pltpu.sync_copy(idx.at[pl.ds(_LAST_BASE, _LAST_VALID)],
                        idx_v.at[pl.ds(0, _LAST_VALID)])
        zeros = jnp.zeros((16,), jnp.int32)
        for z in range(_LAST_VALID, _CHUNK, 16):
            idx_v[pl.ds(z, 16)] = zeros

    ct.wait()
    cr.wait()

    def tile_body(t, carry):
        tbase = t * 512
        gbase = t * 128
        for s in range(8):
            sp3 = idx_v[pl.ds(gbase + s * 16, 16)] * 3
            off = tbase + s * 16
            for tab_v, out_v in ((tab_t_v, out_t_v), (tab_r_v, out_r_v)):
                for c in range(3):
                    vals = plsc.load_gather(tab_v, [sp3 + c])
                    out_v[pl.ds(off + c * 128, 16)] = vals
        return carry

    lax.fori_loop(0, _TPW, tile_body, 0)

    obase = base * 4
    pltpu.sync_copy(out_t_v, out_t.at[pl.ds(obase, _WSTAGE)])
    pltpu.sync_copy(out_r_v, out_r.at[pl.ds(obase, _WSTAGE)])


def _untile(flat):
    tiles = flat.reshape(_TILES, 4, _LANE)
    return tiles.transpose(0, 2, 1).reshape(_TILES * _LANE, 4)[:_N, :3]


def kernel(sp_delta_t, sp_delta_r, p2sp):
    mesh = plsc.VectorSubcoreMesh(core_axis_name="c", subcore_axis_name="s")
    run = pl.kernel(
        _gather_body,
        mesh=mesh,
        compiler_params=pltpu.CompilerParams(needs_layout_passes=False),
        out_type=(
            jax.ShapeDtypeStruct((_OUT_FLAT,), jnp.float32),
            jax.ShapeDtypeStruct((_OUT_FLAT,), jnp.float32),
        ),
        scratch_types=[
            pltpu.VMEM((_NUM_SP * 3,), jnp.float32),
            pltpu.VMEM((_NUM_SP * 3,), jnp.float32),
            pltpu.VMEM((_CHUNK,), jnp.int32),
            pltpu.VMEM((_WSTAGE,), jnp.float32),
            pltpu.VMEM((_WSTAGE,), jnp.float32),
            pltpu.SemaphoreType.DMA,
        ],
    )
    flat_t, flat_r = run(sp_delta_t.reshape(-1), sp_delta_r.reshape(-1), p2sp)
    return _untile(flat_t), _untile(flat_r)


# speedup vs baseline: 7.9081x; 1.1715x over previous
import functools

import jax
import jax.numpy as jnp
from jax import lax
from jax.experimental import pallas as pl
from jax.experimental.pallas import tpu as pltpu
from jax.experimental.pallas import tpu_sc as plsc

_N = 100000
_NUM_SP = 1024
_LANE = 128
_TILES = 782
_OUT_FLAT = _TILES * 4 * _LANE
_NUM_CORES = 2
_NW = 32
_TPW = 25
_CHUNK = _TPW * _LANE
_WSTAGE = _TPW * 4 * _LANE
_LAST_BASE = _TILES * _LANE - _CHUNK
_LAST_VALID = _N - _LAST_BASE


def _gather_body(tab_t, tab_r, idx, out_t, out_r,
                 tab_t_v, tab_r_v, idx_v, out_t_v, out_r_v, sem):
    wid = lax.axis_index("s") * _NUM_CORES + lax.axis_index("c")
    is_tail = wid == _NW - 1
    base = jnp.where(is_tail, _LAST_BASE, wid * _CHUNK)

    ct = pltpu.async_copy(tab_t, tab_t_v, sem)
    cr = pltpu.async_copy(tab_r, tab_r_v, sem)

    @pl.when(jnp.logical_not(is_tail))
    def _():
        pltpu.sync_copy(idx.at[pl.ds(base, _CHUNK)], idx_v.at[pl.ds(0, _CHUNK)])

    @pl.when(is_tail)
    def _():
        pltpu.sync_copy(idx.at[pl.ds(_LAST_BASE, _LAST_VALID)],
                        idx_v.at[pl.ds(0, _LAST_VALID)])
        zeros = jnp.zeros((16,), jnp.int32)
        for z in range(_LAST_VALID, _CHUNK, 16):
            idx_v[pl.ds(z, 16)] = zeros

    ct.wait()
    cr.wait()

    def tile_body(t, carry):
        tbase = t * 512
        gbase = t * 128
        for h in range(2):
            sp3s = [idx_v[pl.ds(gbase + (h * 4 + s) * 16, 16)] * 3
                    for s in range(4)]
            vals = [plsc.load_gather(tab_v, [sp3 + c])
                    for sp3 in sp3s
                    for tab_v in (tab_t_v, tab_r_v)
                    for c in range(3)]
            i = 0
            for s in range(4):
                off = tbase + (h * 4 + s) * 16
                for out_v in (out_t_v, out_r_v):
                    for c in range(3):
                        out_v[pl.ds(off + c * 128, 16)] = vals[i]
                        i += 1
        return carry

    lax.fori_loop(0, _TPW, tile_body, 0)

    obase = base * 4
    pltpu.sync_copy(out_t_v, out_t.at[pl.ds(obase, _WSTAGE)])
    pltpu.sync_copy(out_r_v, out_r.at[pl.ds(obase, _WSTAGE)])


def _untile(flat):
    tiles = flat.reshape(_TILES, 4, _LANE)
    return tiles.transpose(0, 2, 1).reshape(_TILES * _LANE, 4)[:_N, :3]


def kernel(sp_delta_t, sp_delta_r, p2sp):
    mesh = plsc.VectorSubcoreMesh(core_axis_name="c", subcore_axis_name="s")
    run = pl.kernel(
        _gather_body,
        mesh=mesh,
        compiler_params=pltpu.CompilerParams(needs_layout_passes=False),
        out_type=(
            jax.ShapeDtypeStruct((_OUT_FLAT,), jnp.float32),
            jax.ShapeDtypeStruct((_OUT_FLAT,), jnp.float32),
        ),
        scratch_types=[
            pltpu.VMEM((_NUM_SP * 3,), jnp.float32),
            pltpu.VMEM((_NUM_SP * 3,), jnp.float32),
            pltpu.VMEM((_CHUNK,), jnp.int32),
            pltpu.VMEM((_WSTAGE,), jnp.float32),
            pltpu.VMEM((_WSTAGE,), jnp.float32),
            pltpu.SemaphoreType.DMA,
        ],
    )
    flat_t, flat_r = run(sp_delta_t.reshape(-1), sp_delta_r.reshape(-1), p2sp)
    return _untile(flat_t), _untile(flat_r)


# speedup vs baseline: 7.9586x; 1.0064x over previous
import functools

import jax
import jax.numpy as jnp
from jax import lax
from jax.experimental import pallas as pl
from jax.experimental.pallas import tpu as pltpu
from jax.experimental.pallas import tpu_sc as plsc

_N = 100000
_NUM_SP = 1024
_LANE = 128
_TILES = 782
_OUT_FLAT = _TILES * 4 * _LANE
_NUM_CORES = 2
_NW = 32
_TPW = 25
_CHUNK = _TPW * _LANE
_WSTAGE = _TPW * 4 * _LANE
_LAST_BASE = _TILES * _LANE - _CHUNK
_LAST_VALID = _N - _LAST_BASE


def _gather_body(tab_t, tab_r, idx, out_t, out_r,
                 tab_t_v, tab_r_v, idx_v, out_t_v, out_r_v, sem):
    wid = lax.axis_index("s") * _NUM_CORES + lax.axis_index("c")
    is_tail = wid == _NW - 1
    base = jnp.where(is_tail, _LAST_BASE, wid * _CHUNK)

    ct = pltpu.async_copy(tab_t, tab_t_v, sem)
    cr = pltpu.async_copy(tab_r, tab_r_v, sem)

    @pl.when(jnp.logical_not(is_tail))
    def _():
        pltpu.sync_copy(idx.at[pl.ds(base, _CHUNK)], idx_v.at[pl.ds(0, _CHUNK)])

    @pl.when(is_tail)
    def _():
        pltpu.sync_copy(idx.at[pl.ds(_LAST_BASE, _LAST_VALID)],
                        idx_v.at[pl.ds(0, _LAST_VALID)])
        zeros = jnp.zeros((16,), jnp.int32)
        for z in range(_LAST_VALID, _CHUNK, 16):
            idx_v[pl.ds(z, 16)] = zeros

    ct.wait()
    cr.wait()

    def tile_body(t, carry):
        tbase = t * 512
        gbase = t * 128
        for h in range(1):
            sp3s = [idx_v[pl.ds(gbase + (h * 8 + s) * 16, 16)] * 3
                    for s in range(8)]
            vals = [plsc.load_gather(tab_v, [sp3 + c])
                    for sp3 in sp3s
                    for tab_v in (tab_t_v, tab_r_v)
                    for c in range(3)]
            i = 0
            for s in range(8):
                off = tbase + (h * 8 + s) * 16
                for out_v in (out_t_v, out_r_v):
                    for c in range(3):
                        out_v[pl.ds(off + c * 128, 16)] = vals[i]
                        i += 1
        return carry

    lax.fori_loop(0, _TPW, tile_body, 0)

    obase = base * 4
    pltpu.sync_copy(out_t_v, out_t.at[pl.ds(obase, _WSTAGE)])
    pltpu.sync_copy(out_r_v, out_r.at[pl.ds(obase, _WSTAGE)])


def _untile(flat):
    tiles = flat.reshape(_TILES, 4, _LANE)
    return tiles.transpose(0, 2, 1).reshape(_TILES * _LANE, 4)[:_N, :3]


def kernel(sp_delta_t, sp_delta_r, p2sp):
    mesh = plsc.VectorSubcoreMesh(core_axis_name="c", subcore_axis_name="s")
    run = pl.kernel(
        _gather_body,
        mesh=mesh,
        compiler_params=pltpu.CompilerParams(needs_layout_passes=False),
        out_type=(
            jax.ShapeDtypeStruct((_OUT_FLAT,), jnp.float32),
            jax.ShapeDtypeStruct((_OUT_FLAT,), jnp.float32),
        ),
        scratch_types=[
            pltpu.VMEM((_NUM_SP * 3,), jnp.float32),
            pltpu.VMEM((_NUM_SP * 3,), jnp.float32),
            pltpu.VMEM((_CHUNK,), jnp.int32),
            pltpu.VMEM((_WSTAGE,), jnp.float32),
            pltpu.VMEM((_WSTAGE,), jnp.float32),
            pltpu.SemaphoreType.DMA,
        ],
    )
    flat_t, flat_r = run(sp_delta_t.reshape(-1), sp_delta_r.reshape(-1), p2sp)
    return _untile(flat_t), _untile(flat_r)
